# Initial kernel scaffold; baseline (speedup 1.0000x reference)
import jax
import jax.numpy as jnp
from jax.experimental import pallas as pl


def kernel(x, edge_index, W1, a_src1, a_dst1, b1, W_mu, a_src_mu, a_dst_mu, b_mu, W_ls, a_src_ls, a_dst_ls, b_ls):
    raise NotImplementedError("write your pallas kernel here")


# speedup vs baseline: 1.2147x; 1.2147x over previous
import functools
import jax
import jax.numpy as jnp
from jax.experimental import pallas as pl

_N = 10000
_D = 128
_HID = 256
_HEADS = 4
_LAT = 64


def _mm_body(x_ref, w_ref, o_ref):
    o_ref[...] = jnp.dot(x_ref[...], w_ref[...],
                         preferred_element_type=jnp.float32)


def _pl_matmul(x, w, block_m=512):
    m, k = x.shape
    k2, n = w.shape
    assert k == k2
    grid = (m + block_m - 1) // block_m
    return pl.pallas_call(
        _mm_body,
        grid=(grid,),
        in_specs=[
            pl.BlockSpec((block_m, k), lambda i: (i, 0)),
            pl.BlockSpec((k, n), lambda i: (0, 0)),
        ],
        out_specs=pl.BlockSpec((block_m, n), lambda i: (i, 0)),
        out_shape=jax.ShapeDtypeStruct((m, n), jnp.float32),
    )(x, w)


def kernel(x, edge_index, W1, a_src1, a_dst1, b1, W_mu, a_src_mu, a_dst_mu,
           b_mu, W_ls, a_src_ls, a_dst_ls, b_ls):
    n = x.shape[0]
    loops = jnp.arange(n, dtype=edge_index.dtype)
    src = jnp.concatenate([edge_index[0], loops])
    dst = jnp.concatenate([edge_index[1], loops])

    W1r = W1.reshape(_D, _HEADS, _HID)
    Ws1 = jnp.einsum('dhc,hc->dh', W1r, a_src1)
    Wd1 = jnp.einsum('dhc,hc->dh', W1r, a_dst1)
    Wcat = jnp.concatenate([W1, Ws1, Wd1], axis=1)

    out1 = _pl_matmul(x, Wcat)
    h = out1[:, :_HEADS * _HID]
    as1 = out1[:, _HEADS * _HID:_HEADS * _HID + _HEADS]
    ad1 = out1[:, _HEADS * _HID + _HEADS:]

    def agg(h_feat, as_n, ad_n, heads):
        gmax = jax.nn.leaky_relu(
            jnp.max(as_n, axis=0) + jnp.max(ad_n, axis=0),
            negative_slope=0.2)
        e = jax.nn.leaky_relu(as_n[src] + ad_n[dst], negative_slope=0.2)
        ee = jnp.exp(e - gmax[None, :])
        den = jax.ops.segment_sum(ee, dst, num_segments=n)
        outw = jax.ops.segment_sum(h_feat[src] * ee[:, :, None], dst,
                                   num_segments=n)
        return outw / (den[:, :, None] + 1e-16)

    h1 = jax.nn.elu(
        agg(h.reshape(n, _HEADS, _HID), as1, ad1, _HEADS)
        .reshape(n, _HEADS * _HID) + b1)

    Cmu_s = W_mu @ a_src_mu[0]
    Cmu_d = W_mu @ a_dst_mu[0]
    Cls_s = W_ls @ a_src_ls[0]
    Cls_d = W_ls @ a_dst_ls[0]
    W2cat = jnp.concatenate(
        [W_mu, W_ls,
         Cmu_s[:, None], Cmu_d[:, None], Cls_s[:, None], Cls_d[:, None]],
        axis=1)
    out2 = _pl_matmul(h1, W2cat)
    h2m = out2[:, :_LAT]
    h2l = out2[:, _LAT:2 * _LAT]
    sd2 = out2[:, 2 * _LAT:]

    mu = agg(h2m[:, None, :], sd2[:, 0:1], sd2[:, 1:2], 1)[:, 0, :] + b_mu
    logstd = agg(h2l[:, None, :], sd2[:, 2:3], sd2[:, 3:4], 1)[:, 0, :] + b_ls
    return (mu, mu, logstd)


# speedup vs baseline: 4.5686x; 3.7611x over previous
import functools
import jax
import jax.numpy as jnp
from jax import lax
from jax.experimental import pallas as pl
from jax.experimental.pallas import tpu as pltpu
from jax.experimental.pallas import tpu_sc as plsc

_N = 10000
_D = 128
_HID = 256
_HEADS = 4
_LAT = 64

_NT = 16
_CH = 128
_SB = 27
_NSB = 6
_NCH = _SB * _NSB
_EPS = _CH * _SB
_EPT = _CH * _NCH
_EP = _EPT * _NT
_NP = 10240
_NPT = _NP // _NT


def _mm_body(x_ref, w_ref, o_ref):
    o_ref[...] = jnp.dot(x_ref[...], w_ref[...],
                         preferred_element_type=jnp.float32)


def _pl_matmul(x, w, block_m=512):
    m, k = x.shape
    k2, n = w.shape
    grid = (m + block_m - 1) // block_m
    return pl.pallas_call(
        _mm_body,
        grid=(grid,),
        in_specs=[
            pl.BlockSpec((block_m, k), lambda i: (i, 0)),
            pl.BlockSpec((k, n), lambda i: (0, 0)),
        ],
        out_specs=pl.BlockSpec((block_m, n), lambda i: (i, 0)),
        out_shape=jax.ShapeDtypeStruct((m, n), jnp.float32),
    )(x, w)


def _scale_rows(rows_v, ee_v, g, nrows, width):
    def row_body(i, carry):
        ev = plsc.load_gather(
            ee_v, [jnp.broadcast_to(g * _CH + i, (16,))])
        for u in range(width // 16):
            s = pl.ds(u * 16, 16)
            rows_v[i, s] = rows_v[i, s] * ev
        return carry
    lax.fori_loop(0, nrows, row_body, 0)


def _agg_one_slice(h_hbm, out_hbm, ee_hbm, t, src_h, dst_h, src_v, dst_v,
                   ee_v, rows_v, acc, zeros, width):
    pltpu.sync_copy(zeros.at[pl.ds(t * _NPT, _NPT)],
                    acc.at[pl.ds(t * _NPT, _NPT)])
    plsc.subcore_barrier()

    def super_body(ss, carry):
        pltpu.sync_copy(src_h.at[t, ss], src_v)
        pltpu.sync_copy(dst_h.at[t, ss], dst_v)
        pltpu.sync_copy(ee_hbm.at[t, ss], ee_v)

        def chunk_body(b, carry2):
            pltpu.sync_copy(h_hbm.at[src_v.at[b]], rows_v)
            _scale_rows(rows_v, ee_v, b, _CH, width)
            pltpu.sync_copy(rows_v, acc.at[dst_v.at[b]], add=True)
            return carry2
        lax.fori_loop(0, _SB, chunk_body, 0)
        return carry
    lax.fori_loop(0, _NSB, super_body, 0)

    plsc.subcore_barrier()
    pltpu.sync_copy(acc.at[pl.ds(t * _NPT, _NPT)],
                    out_hbm.at[pl.ds(t * _NPT, _NPT)])
    plsc.subcore_barrier()


def _sc_agg1(h_slices, ee_heads, src3, dst3, zeros):
    mesh = plsc.VectorSubcoreMesh(core_axis_name="c", subcore_axis_name="s")

    @functools.partial(
        pl.kernel,
        out_type=[jax.ShapeDtypeStruct((_NP, 128), jnp.float32)] * 8,
        mesh=mesh,
        compiler_params=pltpu.CompilerParams(needs_layout_passes=False),
        scratch_types=[
            pltpu.VMEM((_SB, _CH), jnp.int32),
            pltpu.VMEM((_SB, _CH), jnp.int32),
            pltpu.VMEM((_EPS,), jnp.float32),
            pltpu.VMEM((_CH, 128), jnp.float32),
            pltpu.VMEM_SHARED((_NP, 128), jnp.float32),
        ],
    )
    def k(h0, h1, h2, h3, h4, h5, h6, h7, ee0, ee1, ee2, ee3, src_h, dst_h,
          z_h, o0, o1, o2, o3, o4, o5, o6, o7, src_v, dst_v, ee_v, rows_v,
          acc):
        hs = [h0, h1, h2, h3, h4, h5, h6, h7]
        ees = [ee0, ee1, ee2, ee3]
        os = [o0, o1, o2, o3, o4, o5, o6, o7]
        c = lax.axis_index("c")
        t = lax.axis_index("s")
        for cc in range(2):
            @pl.when(c == cc)
            def _(cc=cc):
                for kk in range(4):
                    sl = cc * 4 + kk
                    _agg_one_slice(hs[sl], os[sl], ees[sl // 2], t, src_h,
                                   dst_h, src_v, dst_v, ee_v, rows_v, acc,
                                   z_h, 128)

    return k(*h_slices, *ee_heads, src3, dst3, zeros)


def _scale_rows_dual(rows_v, eem_v, eel_v, b):
    def row_body(i, carry):
        em = plsc.load_gather(eem_v, [jnp.broadcast_to(b * _CH + i, (16,))])
        el = plsc.load_gather(eel_v, [jnp.broadcast_to(b * _CH + i, (16,))])
        for u in range(4):
            sm = pl.ds(u * 16, 16)
            rows_v[i, sm] = rows_v[i, sm] * em
        for u in range(4, 8):
            sl2 = pl.ds(u * 16, 16)
            rows_v[i, sl2] = rows_v[i, sl2] * el
        return carry
    lax.fori_loop(0, _CH, row_body, 0)


def _sc_agg2(h2, eem, eel, src4, dst4, zeros):
    mesh = plsc.VectorSubcoreMesh(core_axis_name="c", subcore_axis_name="s")

    @functools.partial(
        pl.kernel,
        out_type=[jax.ShapeDtypeStruct((_NP, 128), jnp.float32)] * 2,
        mesh=mesh,
        compiler_params=pltpu.CompilerParams(needs_layout_passes=False),
        scratch_types=[
            pltpu.VMEM((_SB, _CH), jnp.int32),
            pltpu.VMEM((_SB, _CH), jnp.int32),
            pltpu.VMEM((_EPS,), jnp.float32),
            pltpu.VMEM((_EPS,), jnp.float32),
            pltpu.VMEM((_CH, 128), jnp.float32),
            pltpu.VMEM_SHARED((_NP, 128), jnp.float32),
        ],
    )
    def k(h2_h, eem_h, eel_h, src_h, dst_h, z_h, o0, o1, src_v, dst_v,
          eem_v, eel_v, rows_v, acc):
        c = lax.axis_index("c")
        t = lax.axis_index("s")
        w = c * _NT + t
        pltpu.sync_copy(z_h.at[pl.ds(t * _NPT, _NPT)],
                        acc.at[pl.ds(t * _NPT, _NPT)])
        plsc.subcore_barrier()

        def super_body(ss, carry):
            pltpu.sync_copy(src_h.at[w, ss], src_v)
            pltpu.sync_copy(dst_h.at[w, ss], dst_v)
            pltpu.sync_copy(eem_h.at[w, ss], eem_v)
            pltpu.sync_copy(eel_h.at[w, ss], eel_v)

            def chunk_body(b, carry2):
                pltpu.sync_copy(h2_h.at[src_v.at[b]], rows_v)
                _scale_rows_dual(rows_v, eem_v, eel_v, b)
                pltpu.sync_copy(rows_v, acc.at[dst_v.at[b]], add=True)
                return carry2
            lax.fori_loop(0, _SB, chunk_body, 0)
            return carry
        lax.fori_loop(0, _NSB // 2, super_body, 0)

        plsc.subcore_barrier()
        @pl.when(c == 0)
        def _():
            pltpu.sync_copy(acc.at[pl.ds(t * _NPT, _NPT)],
                            o0.at[pl.ds(t * _NPT, _NPT)])
        @pl.when(c == 1)
        def _():
            pltpu.sync_copy(acc.at[pl.ds(t * _NPT, _NPT)],
                            o1.at[pl.ds(t * _NPT, _NPT)])
        plsc.subcore_barrier()

    return k(h2, eem, eel, src4, dst4, zeros)


def _pad_ee(ee_col, e_tot):
    p = jnp.zeros((_EP,), jnp.float32).at[:e_tot].set(ee_col)
    return p.reshape(_NT, _NSB, _EPS)


def kernel(x, edge_index, W1, a_src1, a_dst1, b1, W_mu, a_src_mu, a_dst_mu,
           b_mu, W_ls, a_src_ls, a_dst_ls, b_ls):
    n = x.shape[0]
    loops = jnp.arange(n, dtype=edge_index.dtype)
    src = jnp.concatenate([edge_index[0], loops])
    dst = jnp.concatenate([edge_index[1], loops])
    e_tot = src.shape[0]

    srcp = jnp.zeros((_EP,), jnp.int32).at[:e_tot].set(src)
    dstp = jnp.zeros((_EP,), jnp.int32).at[:e_tot].set(dst)
    src3 = srcp.reshape(_NT, _NSB, _SB, _CH)
    dst3 = dstp.reshape(_NT, _NSB, _SB, _CH)
    zeros = jnp.zeros((_NP, 128), jnp.float32)

    W1r = W1.reshape(_D, _HEADS, _HID)
    Ws1 = jnp.einsum('dhc,hc->dh', W1r, a_src1)
    Wd1 = jnp.einsum('dhc,hc->dh', W1r, a_dst1)
    Wcat = jnp.concatenate([W1, Ws1, Wd1], axis=1)
    out1 = _pl_matmul(x, Wcat)
    h = out1[:, :_HEADS * _HID]
    as1 = out1[:, _HEADS * _HID:_HEADS * _HID + _HEADS]
    ad1 = out1[:, _HEADS * _HID + _HEADS:]

    def scores(as_n, ad_n):
        gmax = jax.nn.leaky_relu(
            jnp.max(as_n, axis=0) + jnp.max(ad_n, axis=0),
            negative_slope=0.2)
        e = jax.nn.leaky_relu(as_n[src] + ad_n[dst], negative_slope=0.2)
        ee = jnp.exp(e - gmax[None, :])
        den = jax.ops.segment_sum(ee, dst, num_segments=n)
        return ee, den

    ee1, den1 = scores(as1, ad1)
    h_slices = [h[:, i * 128:(i + 1) * 128] for i in range(8)]
    ee_heads = [_pad_ee(ee1[:, hd], e_tot) for hd in range(_HEADS)]
    outs = _sc_agg1(h_slices, ee_heads, src3, dst3, zeros)
    outw = jnp.concatenate([o[:n] for o in outs], axis=1).reshape(n, _HEADS, _HID)
    h1 = jax.nn.elu(
        (outw / (den1[:, :, None] + 1e-16)).reshape(n, _HEADS * _HID) + b1)

    Cmu_s = W_mu @ a_src_mu[0]
    Cmu_d = W_mu @ a_dst_mu[0]
    Cls_s = W_ls @ a_src_ls[0]
    Cls_d = W_ls @ a_dst_ls[0]
    W2cat = jnp.concatenate(
        [W_mu, W_ls,
         Cmu_s[:, None], Cmu_d[:, None], Cls_s[:, None], Cls_d[:, None]],
        axis=1)
    out2 = _pl_matmul(h1, W2cat)
    h2m = out2[:, :_LAT]
    h2l = out2[:, _LAT:2 * _LAT]
    sd2 = out2[:, 2 * _LAT:]

    eem, denm = scores(sd2[:, 0:1], sd2[:, 1:2])
    eel, denl = scores(sd2[:, 2:3], sd2[:, 3:4])
    src4 = srcp.reshape(2 * _NT, _NSB // 2, _SB, _CH)
    dst4 = dstp.reshape(2 * _NT, _NSB // 2, _SB, _CH)
    eem4 = _pad_ee(eem[:, 0], e_tot).reshape(2 * _NT, _NSB // 2, _EPS)
    eel4 = _pad_ee(eel[:, 0], e_tot).reshape(2 * _NT, _NSB // 2, _EPS)
    o0, o1 = _sc_agg2(out2[:, :2 * _LAT], eem4, eel4, src4, dst4, zeros)
    out2agg = (o0 + o1)[:n]
    mu = out2agg[:, :_LAT] / (denm + 1e-16) + b_mu
    logstd = out2agg[:, _LAT:] / (denl + 1e-16) + b_ls
    return (mu, mu, logstd)


# speedup vs baseline: 7.1876x; 1.5733x over previous
import functools
import jax
import jax.numpy as jnp
from jax import lax
from jax.experimental import pallas as pl
from jax.experimental.pallas import tpu as pltpu
from jax.experimental.pallas import tpu_sc as plsc

_N = 10000
_D = 128
_HID = 256
_HEADS = 4
_LAT = 64

_NT = 16
_CH = 128
_SB = 27
_NSB = 6
_NCH = _SB * _NSB
_EPS = _CH * _SB
_EPT = _CH * _NCH
_EP = _EPT * _NT
_NP = 10240
_NPT = _NP // _NT


def _mm_body(x_ref, w_ref, o_ref):
    o_ref[...] = jnp.dot(x_ref[...], w_ref[...],
                         preferred_element_type=jnp.float32)


def _pl_matmul(x, w, block_m=512):
    m, k = x.shape
    k2, n = w.shape
    grid = (m + block_m - 1) // block_m
    return pl.pallas_call(
        _mm_body,
        grid=(grid,),
        in_specs=[
            pl.BlockSpec((block_m, k), lambda i: (i, 0)),
            pl.BlockSpec((k, n), lambda i: (0, 0)),
        ],
        out_specs=pl.BlockSpec((block_m, n), lambda i: (i, 0)),
        out_shape=jax.ShapeDtypeStruct((m, n), jnp.float32),
    )(x, w)


def _scale_rows(rows_v, ee_v, g, nrows, width):
    def row_body(i, carry):
        ev = plsc.load_gather(
            ee_v, [jnp.broadcast_to(g * _CH + i, (16,))])
        for u in range(width // 16):
            s = pl.ds(u * 16, 16)
            rows_v[i, s] = rows_v[i, s] * ev
        return carry
    lax.fori_loop(0, nrows, row_body, 0)


def _agg_one_slice(h_hbm, out_hbm, ee_hbm, t, src_h, dst_h, src_v, dst_v,
                   ee_v, rows_v, acc, zeros, width):
    pltpu.sync_copy(zeros.at[pl.ds(t * _NPT, _NPT)],
                    acc.at[pl.ds(t * _NPT, _NPT)])
    plsc.subcore_barrier()

    def super_body(ss, carry):
        pltpu.sync_copy(src_h.at[t, ss], src_v)
        pltpu.sync_copy(dst_h.at[t, ss], dst_v)
        pltpu.sync_copy(ee_hbm.at[t, ss], ee_v)

        def chunk_body(b, carry2):
            pltpu.sync_copy(h_hbm.at[src_v.at[b]], rows_v)
            _scale_rows(rows_v, ee_v, b, _CH, width)
            pltpu.sync_copy(rows_v, acc.at[dst_v.at[b]], add=True)
            return carry2
        lax.fori_loop(0, _SB, chunk_body, 0)
        return carry
    lax.fori_loop(0, _NSB, super_body, 0)

    plsc.subcore_barrier()
    pltpu.sync_copy(acc.at[pl.ds(t * _NPT, _NPT)],
                    out_hbm.at[pl.ds(t * _NPT, _NPT)])
    plsc.subcore_barrier()


def _sc_agg1(h_slices, ee_heads, src3, dst3, zeros):
    mesh = plsc.VectorSubcoreMesh(core_axis_name="c", subcore_axis_name="s")

    @functools.partial(
        pl.kernel,
        out_type=[jax.ShapeDtypeStruct((_NP, 128), jnp.float32)] * 8,
        mesh=mesh,
        compiler_params=pltpu.CompilerParams(needs_layout_passes=False),
        scratch_types=[
            pltpu.VMEM((_SB, _CH), jnp.int32),
            pltpu.VMEM((_SB, _CH), jnp.int32),
            pltpu.VMEM((_EPS,), jnp.float32),
            pltpu.VMEM((_CH, 128), jnp.float32),
            pltpu.VMEM_SHARED((_NP, 128), jnp.float32),
        ],
    )
    def k(h0, h1, h2, h3, h4, h5, h6, h7, ee0, ee1, ee2, ee3, src_h, dst_h,
          z_h, o0, o1, o2, o3, o4, o5, o6, o7, src_v, dst_v, ee_v, rows_v,
          acc):
        hs = [h0, h1, h2, h3, h4, h5, h6, h7]
        ees = [ee0, ee1, ee2, ee3]
        os = [o0, o1, o2, o3, o4, o5, o6, o7]
        c = lax.axis_index("c")
        t = lax.axis_index("s")
        for cc in range(2):
            @pl.when(c == cc)
            def _(cc=cc):
                for kk in range(4):
                    sl = cc * 4 + kk
                    _agg_one_slice(hs[sl], os[sl], ees[sl // 2], t, src_h,
                                   dst_h, src_v, dst_v, ee_v, rows_v, acc,
                                   z_h, 128)

    return k(*h_slices, *ee_heads, src3, dst3, zeros)


def _scale_rows_dual(rows_v, eem_v, eel_v, b):
    def row_body(i, carry):
        em = plsc.load_gather(eem_v, [jnp.broadcast_to(b * _CH + i, (16,))])
        el = plsc.load_gather(eel_v, [jnp.broadcast_to(b * _CH + i, (16,))])
        for u in range(4):
            sm = pl.ds(u * 16, 16)
            rows_v[i, sm] = rows_v[i, sm] * em
        for u in range(4, 8):
            sl2 = pl.ds(u * 16, 16)
            rows_v[i, sl2] = rows_v[i, sl2] * el
        return carry
    lax.fori_loop(0, _CH, row_body, 0)


def _sc_agg2(h2, eem, eel, src4, dst4, zeros):
    mesh = plsc.VectorSubcoreMesh(core_axis_name="c", subcore_axis_name="s")

    @functools.partial(
        pl.kernel,
        out_type=[jax.ShapeDtypeStruct((_NP, 128), jnp.float32)] * 2,
        mesh=mesh,
        compiler_params=pltpu.CompilerParams(needs_layout_passes=False),
        scratch_types=[
            pltpu.VMEM((_SB, _CH), jnp.int32),
            pltpu.VMEM((_SB, _CH), jnp.int32),
            pltpu.VMEM((_EPS,), jnp.float32),
            pltpu.VMEM((_EPS,), jnp.float32),
            pltpu.VMEM((_CH, 128), jnp.float32),
            pltpu.VMEM_SHARED((_NP, 128), jnp.float32),
        ],
    )
    def k(h2_h, eem_h, eel_h, src_h, dst_h, z_h, o0, o1, src_v, dst_v,
          eem_v, eel_v, rows_v, acc):
        c = lax.axis_index("c")
        t = lax.axis_index("s")
        w = c * _NT + t
        pltpu.sync_copy(z_h.at[pl.ds(t * _NPT, _NPT)],
                        acc.at[pl.ds(t * _NPT, _NPT)])
        plsc.subcore_barrier()

        def super_body(ss, carry):
            pltpu.sync_copy(src_h.at[w, ss], src_v)
            pltpu.sync_copy(dst_h.at[w, ss], dst_v)
            pltpu.sync_copy(eem_h.at[w, ss], eem_v)
            pltpu.sync_copy(eel_h.at[w, ss], eel_v)

            def chunk_body(b, carry2):
                pltpu.sync_copy(h2_h.at[src_v.at[b]], rows_v)
                _scale_rows_dual(rows_v, eem_v, eel_v, b)
                pltpu.sync_copy(rows_v, acc.at[dst_v.at[b]], add=True)
                return carry2
            lax.fori_loop(0, _SB, chunk_body, 0)
            return carry
        lax.fori_loop(0, _NSB // 2, super_body, 0)

        plsc.subcore_barrier()
        @pl.when(c == 0)
        def _():
            pltpu.sync_copy(acc.at[pl.ds(t * _NPT, _NPT)],
                            o0.at[pl.ds(t * _NPT, _NPT)])
        @pl.when(c == 1)
        def _():
            pltpu.sync_copy(acc.at[pl.ds(t * _NPT, _NPT)],
                            o1.at[pl.ds(t * _NPT, _NPT)])
        plsc.subcore_barrier()

    return k(h2, eem, eel, src4, dst4, zeros)


def _pad_ee(ee_col, e_tot):
    p = jnp.zeros((_EP,), jnp.float32).at[:e_tot].set(ee_col)
    return p.reshape(_NT, _NSB, _EPS)


def kernel(x, edge_index, W1, a_src1, a_dst1, b1, W_mu, a_src_mu, a_dst_mu,
           b_mu, W_ls, a_src_ls, a_dst_ls, b_ls):
    n = x.shape[0]
    loops = jnp.arange(n, dtype=edge_index.dtype)
    src = jnp.concatenate([edge_index[0], loops])
    dst = jnp.concatenate([edge_index[1], loops])
    e_tot = src.shape[0]

    srcp = jnp.zeros((_EP,), jnp.int32).at[:e_tot].set(src)
    dstp = jnp.zeros((_EP,), jnp.int32).at[:e_tot].set(dst)
    src3 = srcp.reshape(_NT, _NSB, _SB, _CH)
    dst3 = dstp.reshape(_NT, _NSB, _SB, _CH)
    zeros = jnp.zeros((_NP, 128), jnp.float32)

    W1r = W1.reshape(_D, _HEADS, _HID)
    Ws1 = jnp.einsum('dhc,hc->dh', W1r, a_src1)
    Wd1 = jnp.einsum('dhc,hc->dh', W1r, a_dst1)
    Wcat = jnp.concatenate([W1, Ws1, Wd1], axis=1)
    out1 = _pl_matmul(x, Wcat)
    h = out1[:, :_HEADS * _HID]
    as1 = out1[:, _HEADS * _HID:_HEADS * _HID + _HEADS]
    ad1 = out1[:, _HEADS * _HID + _HEADS:]

    def scores(as_n, ad_n):
        gmax = jax.nn.leaky_relu(
            jnp.max(as_n, axis=0) + jnp.max(ad_n, axis=0),
            negative_slope=0.2)
        e = jax.nn.leaky_relu(as_n[src] + ad_n[dst], negative_slope=0.2)
        ee = jnp.exp(e - gmax[None, :])
        den = jax.ops.segment_sum(ee, dst, num_segments=n)
        return ee, den

    ee1, den1 = scores(as1, ad1)
    h_slices = [h[:, i * 128:(i + 1) * 128] for i in range(8)]
    ee_heads = [_pad_ee(ee1[:, hd], e_tot) for hd in range(_HEADS)]
    outs = _sc_agg1(h_slices, ee_heads, src3, dst3, zeros)
    outw = jnp.concatenate([o[:n] for o in outs], axis=1).reshape(n, _HEADS, _HID)
    h1 = jax.nn.elu(
        (outw / (den1[:, :, None] + 1e-16)).reshape(n, _HEADS * _HID) + b1)

    Cmu_s = W_mu @ a_src_mu[0]
    Cmu_d = W_mu @ a_dst_mu[0]
    Cls_s = W_ls @ a_src_ls[0]
    Cls_d = W_ls @ a_dst_ls[0]
    W2cat = jnp.concatenate(
        [W_mu, W_ls,
         Cmu_s[:, None], Cls_s[:, None], Cmu_d[:, None], Cls_d[:, None]],
        axis=1)
    out2 = _pl_matmul(h1, W2cat)
    sd2 = out2[:, 2 * _LAT:]

    ee2, den2 = scores(sd2[:, 0:2], sd2[:, 2:4])
    eem, denm = ee2[:, 0:1], den2[:, 0:1]
    eel, denl = ee2[:, 1:2], den2[:, 1:2]
    src4 = srcp.reshape(2 * _NT, _NSB // 2, _SB, _CH)
    dst4 = dstp.reshape(2 * _NT, _NSB // 2, _SB, _CH)
    eem4 = _pad_ee(eem[:, 0], e_tot).reshape(2 * _NT, _NSB // 2, _EPS)
    eel4 = _pad_ee(eel[:, 0], e_tot).reshape(2 * _NT, _NSB // 2, _EPS)
    o0, o1 = _sc_agg2(out2[:, :2 * _LAT], eem4, eel4, src4, dst4, zeros)
    out2agg = (o0 + o1)[:n]
    mu = out2agg[:, :_LAT] / (denm + 1e-16) + b_mu
    logstd = out2agg[:, _LAT:] / (denl + 1e-16) + b_ls
    return (mu, mu, logstd)


# speedup vs baseline: 9.5237x; 1.3250x over previous
import functools
import jax
import jax.numpy as jnp
from jax import lax
from jax.experimental import pallas as pl
from jax.experimental.pallas import tpu as pltpu
from jax.experimental.pallas import tpu_sc as plsc

_N = 10000
_D = 128
_HID = 256
_HEADS = 4
_LAT = 64

_NT = 16
_CH = 128
_SB = 27
_NSB = 6
_NCH = _SB * _NSB
_EPS = _CH * _SB
_EPT = _CH * _NCH
_EP = _EPT * _NT
_NP = 10240
_NPT = _NP // _NT
_SB2 = 9
_NSB2 = 9
_EPS2 = _CH * _SB2


def _mm_body(x_ref, w_ref, o_ref):
    o_ref[...] = jnp.dot(x_ref[...], w_ref[...],
                         preferred_element_type=jnp.float32)


def _pl_matmul(x, w, block_m=512):
    m, k = x.shape
    k2, n = w.shape
    grid = (m + block_m - 1) // block_m
    return pl.pallas_call(
        _mm_body,
        grid=(grid,),
        in_specs=[
            pl.BlockSpec((block_m, k), lambda i: (i, 0)),
            pl.BlockSpec((k, n), lambda i: (0, 0)),
        ],
        out_specs=pl.BlockSpec((block_m, n), lambda i: (i, 0)),
        out_shape=jax.ShapeDtypeStruct((m, n), jnp.float32),
    )(x, w)


def _scale_rows(rows_v, ee_v, g, nrows, width):
    def row_body(i, carry):
        ev = plsc.load_gather(
            ee_v, [jnp.broadcast_to(g * _CH + i, (16,))])
        for u in range(width // 16):
            s = pl.ds(u * 16, 16)
            rows_v[i, s] = rows_v[i, s] * ev
        return carry
    lax.fori_loop(0, nrows, row_body, 0)


def _den_scatter(den_part, dst_v, ee_v, b):
    for u in range(_CH // 16):
        dst16 = dst_v[b, pl.ds(u * 16, 16)]
        vals = ee_v[pl.ds(b * _CH + u * 16, 16)]
        plsc.addupdate_scatter(den_part, [dst16], vals)


def _agg_one_slice(h_hbm, out_hbm, ee_hbm, t, src_h, dst_h, src_v, dst_v,
                   ee_v, rows_v, acc, zeros, width, den_part=None):
    rs = pl.ds(t * _NPT, _NPT)
    pltpu.sync_copy(zeros, acc.at[rs])
    plsc.subcore_barrier()

    def super_body(ss, carry):
        pltpu.sync_copy(src_h.at[t, ss], src_v)
        pltpu.sync_copy(dst_h.at[t, ss], dst_v)
        pltpu.sync_copy(ee_hbm.at[t, ss], ee_v)

        def chunk_body(b, carry2):
            pltpu.sync_copy(h_hbm.at[src_v.at[b]], rows_v)
            _scale_rows(rows_v, ee_v, b, _CH, width)
            pltpu.sync_copy(rows_v, acc.at[dst_v.at[b]], add=True)
            if den_part is not None:
                _den_scatter(den_part, dst_v, ee_v, b)
            return carry2
        lax.fori_loop(0, _SB, chunk_body, 0)
        return carry
    lax.fori_loop(0, _NSB, super_body, 0)

    plsc.subcore_barrier()
    pltpu.sync_copy(acc.at[rs], out_hbm.at[rs])
    plsc.subcore_barrier()


def _sc_agg1(h_slices, ee_heads, src3, dst3, zeros, zeros_np):
    mesh = plsc.VectorSubcoreMesh(core_axis_name="c", subcore_axis_name="s")

    @functools.partial(
        pl.kernel,
        out_type=([jax.ShapeDtypeStruct((_NP, 128), jnp.float32)] * 8
                  + [jax.ShapeDtypeStruct((_NT, _NP), jnp.float32)] * 4),
        mesh=mesh,
        compiler_params=pltpu.CompilerParams(needs_layout_passes=False),
        scratch_types=[
            pltpu.VMEM((_SB, _CH), jnp.int32),
            pltpu.VMEM((_SB, _CH), jnp.int32),
            pltpu.VMEM((_EPS,), jnp.float32),
            pltpu.VMEM((_CH, 128), jnp.float32),
            pltpu.VMEM((_NP,), jnp.float32),
            pltpu.VMEM_SHARED((_NP, 128), jnp.float32),
        ],
    )
    def k(h0, h1, h2, h3, h4, h5, h6, h7, ee0, ee1, ee2, ee3, src_h, dst_h,
          z_h, znp_h, o0, o1, o2, o3, o4, o5, o6, o7, d0, d1, d2, d3,
          src_v, dst_v, ee_v, rows_v, den_p, acc):
        hs = [h0, h1, h2, h3, h4, h5, h6, h7]
        ees = [ee0, ee1, ee2, ee3]
        os = [o0, o1, o2, o3, o4, o5, o6, o7]
        ds = [d0, d1, d2, d3]
        c = lax.axis_index("c")
        t = lax.axis_index("s")
        for cc in range(2):
            @pl.when(c == cc)
            def _(cc=cc):
                for kk in range(4):
                    sl = cc * 4 + kk
                    if kk % 2 == 0:
                        pltpu.sync_copy(znp_h, den_p)
                    _agg_one_slice(hs[sl], os[sl], ees[sl // 2], t, src_h,
                                   dst_h, src_v, dst_v, ee_v, rows_v, acc,
                                   z_h, 128, den_p if kk % 2 == 0 else None)
                    if kk % 2 == 0:
                        pltpu.sync_copy(den_p, ds[cc * 2 + kk // 2].at[t])

    return k(*h_slices, *ee_heads, src3, dst3, zeros, zeros_np)


def _scale_rows_dual(rows_v, eem_v, eel_v, b):
    def row_body(i, carry):
        em = plsc.load_gather(eem_v, [jnp.broadcast_to(b * _CH + i, (16,))])
        el = plsc.load_gather(eel_v, [jnp.broadcast_to(b * _CH + i, (16,))])
        for u in range(4):
            sm = pl.ds(u * 16, 16)
            rows_v[i, sm] = rows_v[i, sm] * em
        for u in range(4, 8):
            sl2 = pl.ds(u * 16, 16)
            rows_v[i, sl2] = rows_v[i, sl2] * el
        return carry
    lax.fori_loop(0, _CH, row_body, 0)


def _sc_agg2(h2, eem, eel, src4, dst4, zeros, zeros_np):
    mesh = plsc.VectorSubcoreMesh(core_axis_name="c", subcore_axis_name="s")

    @functools.partial(
        pl.kernel,
        out_type=([jax.ShapeDtypeStruct((_NP, 128), jnp.float32)] * 2
                  + [jax.ShapeDtypeStruct((2 * _NT, _NP), jnp.float32)] * 2),
        mesh=mesh,
        compiler_params=pltpu.CompilerParams(needs_layout_passes=False),
        scratch_types=[
            pltpu.VMEM((_SB2, _CH), jnp.int32),
            pltpu.VMEM((_SB2, _CH), jnp.int32),
            pltpu.VMEM((_EPS2,), jnp.float32),
            pltpu.VMEM((_EPS2,), jnp.float32),
            pltpu.VMEM((_CH, 128), jnp.float32),
            pltpu.VMEM((_NP,), jnp.float32),
            pltpu.VMEM((_NP,), jnp.float32),
            pltpu.VMEM_SHARED((_NP, 128), jnp.float32),
        ],
    )
    def k(h2_h, eem_h, eel_h, src_h, dst_h, z_h, znp_h, o0, o1, dm, dl,
          src_v, dst_v, eem_v, eel_v, rows_v, den_m, den_l, acc):
        c = lax.axis_index("c")
        t = lax.axis_index("s")
        w = c * _NT + t
        rs = pl.ds(t * _NPT, _NPT)
        pltpu.sync_copy(z_h, acc.at[rs])
        pltpu.sync_copy(znp_h, den_m)
        pltpu.sync_copy(znp_h, den_l)
        plsc.subcore_barrier()

        def super_body(ss, carry):
            pltpu.sync_copy(src_h.at[w, ss], src_v)
            pltpu.sync_copy(dst_h.at[w, ss], dst_v)
            pltpu.sync_copy(eem_h.at[w, ss], eem_v)
            pltpu.sync_copy(eel_h.at[w, ss], eel_v)

            def chunk_body(b, carry2):
                pltpu.sync_copy(h2_h.at[src_v.at[b]], rows_v)
                _scale_rows_dual(rows_v, eem_v, eel_v, b)
                pltpu.sync_copy(rows_v, acc.at[dst_v.at[b]], add=True)
                _den_scatter(den_m, dst_v, eem_v, b)
                _den_scatter(den_l, dst_v, eel_v, b)
                return carry2
            lax.fori_loop(0, _SB2, chunk_body, 0)
            return carry
        lax.fori_loop(0, _NSB2, super_body, 0)

        plsc.subcore_barrier()
        pltpu.sync_copy(den_m, dm.at[w])
        pltpu.sync_copy(den_l, dl.at[w])
        @pl.when(c == 0)
        def _():
            pltpu.sync_copy(acc.at[rs], o0.at[rs])
        @pl.when(c == 1)
        def _():
            pltpu.sync_copy(acc.at[rs], o1.at[rs])
        plsc.subcore_barrier()

    return k(h2, eem, eel, src4, dst4, zeros, zeros_np)


def _pad_ee(ee_col, e_tot):
    p = jnp.zeros((_EP,), jnp.float32).at[:e_tot].set(ee_col)
    return p.reshape(_NT, _NSB, _EPS)


def kernel(x, edge_index, W1, a_src1, a_dst1, b1, W_mu, a_src_mu, a_dst_mu,
           b_mu, W_ls, a_src_ls, a_dst_ls, b_ls):
    n = x.shape[0]
    loops = jnp.arange(n, dtype=edge_index.dtype)
    src = jnp.concatenate([edge_index[0], loops])
    dst = jnp.concatenate([edge_index[1], loops])
    e_tot = src.shape[0]

    srcp = jnp.zeros((_EP,), jnp.int32).at[:e_tot].set(src)
    dstp = jnp.zeros((_EP,), jnp.int32).at[:e_tot].set(dst)
    src3 = srcp.reshape(_NT, _NSB, _SB, _CH)
    dst3 = dstp.reshape(_NT, _NSB, _SB, _CH)
    zeros = jnp.zeros((_NPT, 128), jnp.float32)
    zeros_np = jnp.zeros((_NP,), jnp.float32)

    W1r = W1.reshape(_D, _HEADS, _HID)
    Ws1 = jnp.einsum('dhc,hc->dh', W1r, a_src1)
    Wd1 = jnp.einsum('dhc,hc->dh', W1r, a_dst1)
    Wcat = jnp.concatenate([W1, Ws1, Wd1], axis=1)
    out1 = _pl_matmul(x, Wcat)
    h = out1[:, :_HEADS * _HID]
    as1 = out1[:, _HEADS * _HID:_HEADS * _HID + _HEADS]
    ad1 = out1[:, _HEADS * _HID + _HEADS:]

    def scores(as_n, ad_n):
        gmax = jax.nn.leaky_relu(
            jnp.max(as_n, axis=0) + jnp.max(ad_n, axis=0),
            negative_slope=0.2)
        e = jax.nn.leaky_relu(as_n[src] + ad_n[dst], negative_slope=0.2)
        return jnp.exp(e - gmax[None, :])

    ee1 = scores(as1, ad1)
    h_slices = [h[:, i * 128:(i + 1) * 128] for i in range(8)]
    ee_heads = [_pad_ee(ee1[:, hd], e_tot) for hd in range(_HEADS)]
    res1 = _sc_agg1(h_slices, ee_heads, src3, dst3, zeros, zeros_np)
    outs, dens = res1[:8], res1[8:]
    den1 = jnp.stack([d.sum(axis=0)[:n] for d in dens], axis=1)
    outw = jnp.concatenate([o[:n] for o in outs], axis=1).reshape(n, _HEADS, _HID)
    h1 = jax.nn.elu(
        (outw / (den1[:, :, None] + 1e-16)).reshape(n, _HEADS * _HID) + b1)

    Cmu_s = W_mu @ a_src_mu[0]
    Cmu_d = W_mu @ a_dst_mu[0]
    Cls_s = W_ls @ a_src_ls[0]
    Cls_d = W_ls @ a_dst_ls[0]
    W2cat = jnp.concatenate(
        [W_mu, W_ls,
         Cmu_s[:, None], Cls_s[:, None], Cmu_d[:, None], Cls_d[:, None]],
        axis=1)
    out2 = _pl_matmul(h1, W2cat)
    sd2 = out2[:, 2 * _LAT:]

    ee2 = scores(sd2[:, 0:2], sd2[:, 2:4])
    src4 = srcp.reshape(2 * _NT, _NSB2, _SB2, _CH)
    dst4 = dstp.reshape(2 * _NT, _NSB2, _SB2, _CH)
    eem4 = _pad_ee(ee2[:, 0], e_tot).reshape(2 * _NT, _NSB2, _EPS2)
    eel4 = _pad_ee(ee2[:, 1], e_tot).reshape(2 * _NT, _NSB2, _EPS2)
    o0, o1, dm, dl = _sc_agg2(out2[:, :2 * _LAT], eem4, eel4, src4, dst4,
                              zeros, zeros_np)
    out2agg = (o0 + o1)[:n]
    denm = dm.sum(axis=0)[:n, None]
    denl = dl.sum(axis=0)[:n, None]
    mu = out2agg[:, :_LAT] / (denm + 1e-16) + b_mu
    logstd = out2agg[:, _LAT:] / (denl + 1e-16) + b_ls
    return (mu, mu, logstd)


# speedup vs baseline: 19.1054x; 2.0061x over previous
import functools
import jax
import jax.numpy as jnp
from jax import lax
from jax.experimental import pallas as pl
from jax.experimental.pallas import tpu as pltpu
from jax.experimental.pallas import tpu_sc as plsc

_N = 10000
_D = 128
_HID = 256
_HEADS = 4
_LAT = 64

_NT = 16
_CH = 128
_SB = 27
_NSB = 6
_NCH = _SB * _NSB
_EPS = _CH * _SB
_EPT = _CH * _NCH
_EP = _EPT * _NT
_NP = 10240
_NPT = _NP // _NT
_SB2 = 9
_NSB2 = 9
_EPS2 = _CH * _SB2
_SBS = 27
_NSBS = 3
_EPSS = _CH * _SBS


def _mm_body(x_ref, w_ref, o_ref):
    o_ref[...] = jnp.dot(x_ref[...], w_ref[...],
                         preferred_element_type=jnp.float32)


def _pl_matmul(x, w, block_m=512):
    m, k = x.shape
    k2, n = w.shape
    grid = (m + block_m - 1) // block_m
    return pl.pallas_call(
        _mm_body,
        grid=(grid,),
        in_specs=[
            pl.BlockSpec((block_m, k), lambda i: (i, 0)),
            pl.BlockSpec((k, n), lambda i: (0, 0)),
        ],
        out_specs=pl.BlockSpec((block_m, n), lambda i: (i, 0)),
        out_shape=jax.ShapeDtypeStruct((m, n), jnp.float32),
    )(x, w)


def _scale_rows(rows_v, ee_v, g, nrows, width):
    def row_body(i, carry):
        ev = plsc.load_gather(
            ee_v, [jnp.broadcast_to(g * _CH + i, (16,))])
        for u in range(width // 16):
            s = pl.ds(u * 16, 16)
            rows_v[i, s] = rows_v[i, s] * ev
        return carry
    lax.fori_loop(0, nrows, row_body, 0)


def _den_scatter(den_part, dst_v, ee_v, b):
    for u in range(_CH // 16):
        dst16 = dst_v[b, pl.ds(u * 16, 16)]
        vals = ee_v[pl.ds(b * _CH + u * 16, 16)]
        plsc.addupdate_scatter(den_part, [dst16], vals)


def _agg_one_slice(h_hbm, out_hbm, ee_hbm, t, src_h, dst_h, src_v, dst_v,
                   ee_v, rows_v, acc, zeros, width, den_part=None):
    rs = pl.ds(t * _NPT, _NPT)
    pltpu.sync_copy(zeros, acc.at[rs])
    plsc.subcore_barrier()

    def super_body(ss, carry):
        pltpu.sync_copy(src_h.at[t, ss], src_v)
        pltpu.sync_copy(dst_h.at[t, ss], dst_v)
        pltpu.sync_copy(ee_hbm.at[t, ss], ee_v)

        def chunk_body(b, carry2):
            pltpu.sync_copy(h_hbm.at[src_v.at[b]], rows_v)
            _scale_rows(rows_v, ee_v, b, _CH, width)
            pltpu.sync_copy(rows_v, acc.at[dst_v.at[b]], add=True)
            if den_part is not None:
                _den_scatter(den_part, dst_v, ee_v, b)
            return carry2
        lax.fori_loop(0, _SB, chunk_body, 0)
        return carry
    lax.fori_loop(0, _NSB, super_body, 0)

    plsc.subcore_barrier()
    pltpu.sync_copy(acc.at[rs], out_hbm.at[rs])
    plsc.subcore_barrier()


def _sc_agg1(h_slices, ee_heads, src3, dst3, zeros, zeros_np):
    mesh = plsc.VectorSubcoreMesh(core_axis_name="c", subcore_axis_name="s")

    @functools.partial(
        pl.kernel,
        out_type=([jax.ShapeDtypeStruct((_NP, 128), jnp.float32)] * 8
                  + [jax.ShapeDtypeStruct((_NT, _NP), jnp.float32)] * 4),
        mesh=mesh,
        compiler_params=pltpu.CompilerParams(needs_layout_passes=False),
        scratch_types=[
            pltpu.VMEM((_SB, _CH), jnp.int32),
            pltpu.VMEM((_SB, _CH), jnp.int32),
            pltpu.VMEM((_EPS,), jnp.float32),
            pltpu.VMEM((_CH, 128), jnp.float32),
            pltpu.VMEM((_NP,), jnp.float32),
            pltpu.VMEM_SHARED((_NP, 128), jnp.float32),
        ],
    )
    def k(h0, h1, h2, h3, h4, h5, h6, h7, ee0, ee1, ee2, ee3, src_h, dst_h,
          z_h, znp_h, o0, o1, o2, o3, o4, o5, o6, o7, d0, d1, d2, d3,
          src_v, dst_v, ee_v, rows_v, den_p, acc):
        hs = [h0, h1, h2, h3, h4, h5, h6, h7]
        ees = [ee0, ee1, ee2, ee3]
        os = [o0, o1, o2, o3, o4, o5, o6, o7]
        ds = [d0, d1, d2, d3]
        c = lax.axis_index("c")
        t = lax.axis_index("s")
        for cc in range(2):
            @pl.when(c == cc)
            def _(cc=cc):
                for kk in range(4):
                    sl = cc * 4 + kk
                    if kk % 2 == 0:
                        pltpu.sync_copy(znp_h, den_p)
                    _agg_one_slice(hs[sl], os[sl], ees[sl // 2], t, src_h,
                                   dst_h, src_v, dst_v, ee_v, rows_v, acc,
                                   z_h, 128, den_p if kk % 2 == 0 else None)
                    if kk % 2 == 0:
                        pltpu.sync_copy(den_p, ds[cc * 2 + kk // 2].at[t])

    return k(*h_slices, *ee_heads, src3, dst3, zeros, zeros_np)


def _scale_rows_dual(rows_v, eem_v, eel_v, b):
    def row_body(i, carry):
        em = plsc.load_gather(eem_v, [jnp.broadcast_to(b * _CH + i, (16,))])
        el = plsc.load_gather(eel_v, [jnp.broadcast_to(b * _CH + i, (16,))])
        for u in range(4):
            sm = pl.ds(u * 16, 16)
            rows_v[i, sm] = rows_v[i, sm] * em
        for u in range(4, 8):
            sl2 = pl.ds(u * 16, 16)
            rows_v[i, sl2] = rows_v[i, sl2] * el
        return carry
    lax.fori_loop(0, _CH, row_body, 0)


def _sc_agg2(h2, eem, eel, src4, dst4, zeros, zeros_np):
    mesh = plsc.VectorSubcoreMesh(core_axis_name="c", subcore_axis_name="s")

    @functools.partial(
        pl.kernel,
        out_type=([jax.ShapeDtypeStruct((_NP, 128), jnp.float32)] * 2
                  + [jax.ShapeDtypeStruct((2 * _NT, _NP), jnp.float32)] * 2),
        mesh=mesh,
        compiler_params=pltpu.CompilerParams(needs_layout_passes=False),
        scratch_types=[
            pltpu.VMEM((_SB2, _CH), jnp.int32),
            pltpu.VMEM((_SB2, _CH), jnp.int32),
            pltpu.VMEM((_EPS2,), jnp.float32),
            pltpu.VMEM((_EPS2,), jnp.float32),
            pltpu.VMEM((_CH, 128), jnp.float32),
            pltpu.VMEM((_NP,), jnp.float32),
            pltpu.VMEM((_NP,), jnp.float32),
            pltpu.VMEM_SHARED((_NP, 128), jnp.float32),
        ],
    )
    def k(h2_h, eem_h, eel_h, src_h, dst_h, z_h, znp_h, o0, o1, dm, dl,
          src_v, dst_v, eem_v, eel_v, rows_v, den_m, den_l, acc):
        c = lax.axis_index("c")
        t = lax.axis_index("s")
        w = c * _NT + t
        rs = pl.ds(t * _NPT, _NPT)
        pltpu.sync_copy(z_h, acc.at[rs])
        pltpu.sync_copy(znp_h, den_m)
        pltpu.sync_copy(znp_h, den_l)
        plsc.subcore_barrier()

        def super_body(ss, carry):
            pltpu.sync_copy(src_h.at[w, ss], src_v)
            pltpu.sync_copy(dst_h.at[w, ss], dst_v)
            pltpu.sync_copy(eem_h.at[w, ss], eem_v)
            pltpu.sync_copy(eel_h.at[w, ss], eel_v)

            def chunk_body(b, carry2):
                pltpu.sync_copy(h2_h.at[src_v.at[b]], rows_v)
                _scale_rows_dual(rows_v, eem_v, eel_v, b)
                pltpu.sync_copy(rows_v, acc.at[dst_v.at[b]], add=True)
                _den_scatter(den_m, dst_v, eem_v, b)
                _den_scatter(den_l, dst_v, eel_v, b)
                return carry2
            lax.fori_loop(0, _SB2, chunk_body, 0)
            return carry
        lax.fori_loop(0, _NSB2, super_body, 0)

        plsc.subcore_barrier()
        pltpu.sync_copy(den_m, dm.at[w])
        pltpu.sync_copy(den_l, dl.at[w])
        @pl.when(c == 0)
        def _():
            pltpu.sync_copy(acc.at[rs], o0.at[rs])
        @pl.when(c == 1)
        def _():
            pltpu.sync_copy(acc.at[rs], o1.at[rs])
        plsc.subcore_barrier()

    return k(h2, eem, eel, src4, dst4, zeros, zeros_np)


def _sc_scores(tabs, tabd, gbc, src2, dst2, nheads):
    mesh = plsc.VectorSubcoreMesh(core_axis_name="c", subcore_axis_name="s")
    scratch = ([
        pltpu.VMEM((_SBS, _CH), jnp.int32),
        pltpu.VMEM((_SBS, _CH), jnp.int32),
        pltpu.VMEM((nheads * 16,), jnp.float32),
    ] + [pltpu.VMEM((_NP,), jnp.float32) for _ in range(2 * nheads)]
      + [pltpu.VMEM((_EPSS,), jnp.float32) for _ in range(nheads)])

    @functools.partial(
        pl.kernel,
        out_type=[jax.ShapeDtypeStruct((2 * _NT, _NSBS, _EPSS), jnp.float32)
                  for _ in range(nheads)],
        mesh=mesh,
        compiler_params=pltpu.CompilerParams(needs_layout_passes=False),
        scratch_types=scratch,
    )
    def k(tabs_h, tabd_h, g_h, src_h, dst_h, *rest):
        outs = rest[:nheads]
        src_v, dst_v, g_v = rest[nheads:nheads + 3]
        tabs_vs = rest[nheads + 3:nheads + 3 + nheads]
        tabd_vs = rest[nheads + 3 + nheads:nheads + 3 + 2 * nheads]
        eebs = rest[nheads + 3 + 2 * nheads:]
        c = lax.axis_index("c")
        t = lax.axis_index("s")
        w = c * _NT + t
        pltpu.sync_copy(g_h, g_v)
        for hd in range(nheads):
            pltpu.sync_copy(tabs_h.at[hd], tabs_vs[hd])
            pltpu.sync_copy(tabd_h.at[hd], tabd_vs[hd])

        def super_body(ss, carry):
            pltpu.sync_copy(src_h.at[w, ss], src_v)
            pltpu.sync_copy(dst_h.at[w, ss], dst_v)

            def chunk_body(b, carry2):
                for u in range(_CH // 16):
                    s16 = src_v[b, pl.ds(u * 16, 16)]
                    d16 = dst_v[b, pl.ds(u * 16, 16)]
                    for hd in range(nheads):
                        s = (plsc.load_gather(tabs_vs[hd], [s16])
                             + plsc.load_gather(tabd_vs[hd], [d16]))
                        lr = jnp.maximum(s, 0.2 * s)
                        ee = jnp.exp(lr - g_v[pl.ds(hd * 16, 16)])
                        eebs[hd][pl.ds(b * _CH + u * 16, 16)] = ee
                return carry2
            lax.fori_loop(0, _SBS, chunk_body, 0)
            for hd in range(nheads):
                pltpu.sync_copy(eebs[hd], outs[hd].at[w, ss])
            return carry
        lax.fori_loop(0, _NSBS, super_body, 0)

    return k(tabs, tabd, gbc, src2, dst2)


def kernel(x, edge_index, W1, a_src1, a_dst1, b1, W_mu, a_src_mu, a_dst_mu,
           b_mu, W_ls, a_src_ls, a_dst_ls, b_ls):
    n = x.shape[0]
    loops = jnp.arange(n, dtype=edge_index.dtype)
    src = jnp.concatenate([edge_index[0], loops])
    dst = jnp.concatenate([edge_index[1], loops])
    e_tot = src.shape[0]

    srcp = jnp.zeros((_EP,), jnp.int32).at[:e_tot].set(src)
    dstp = jnp.full((_EP,), _NP - 1, jnp.int32).at[:e_tot].set(dst)
    src3 = srcp.reshape(_NT, _NSB, _SB, _CH)
    dst3 = dstp.reshape(_NT, _NSB, _SB, _CH)
    srcs = srcp.reshape(2 * _NT, _NSBS, _SBS, _CH)
    dsts = dstp.reshape(2 * _NT, _NSBS, _SBS, _CH)
    zeros = jnp.zeros((_NPT, 128), jnp.float32)
    zeros_np = jnp.zeros((_NP,), jnp.float32)
    neg = jnp.float32(-1e30)

    W1r = W1.reshape(_D, _HEADS, _HID)
    Ws1 = jnp.einsum('dhc,hc->dh', W1r, a_src1)
    Wd1 = jnp.einsum('dhc,hc->dh', W1r, a_dst1)
    Wcat = jnp.concatenate([W1, Ws1, Wd1], axis=1)
    out1 = _pl_matmul(x, Wcat)
    h = out1[:, :_HEADS * _HID]
    as1 = out1[:, _HEADS * _HID:_HEADS * _HID + _HEADS]
    ad1 = out1[:, _HEADS * _HID + _HEADS:]

    def score_prep(as_n, ad_n, nh):
        gmax = jax.nn.leaky_relu(
            jnp.max(as_n, axis=0) + jnp.max(ad_n, axis=0),
            negative_slope=0.2)
        tabs = jnp.zeros((nh, _NP), jnp.float32).at[:, :n].set(as_n.T)
        tabd = jnp.full((nh, _NP), neg, jnp.float32).at[:, :n].set(ad_n.T)
        gbc = jnp.broadcast_to(gmax[:, None], (nh, 16)).reshape(nh * 16)
        return tabs, tabd, gbc

    tabs1, tabd1, g1 = score_prep(as1, ad1, _HEADS)
    eeh1 = _sc_scores(tabs1, tabd1, g1, srcs, dsts, _HEADS)
    h_slices = [h[:, i * 128:(i + 1) * 128] for i in range(8)]
    ee_heads = [r.reshape(_NT, _NSB, _EPS) for r in eeh1]
    res1 = _sc_agg1(h_slices, ee_heads, src3, dst3, zeros, zeros_np)
    outs, dens = res1[:8], res1[8:]
    den1 = jnp.stack([d.sum(axis=0)[:n] for d in dens], axis=1)
    outw = jnp.concatenate([o[:n] for o in outs], axis=1).reshape(n, _HEADS, _HID)
    h1 = jax.nn.elu(
        (outw / (den1[:, :, None] + 1e-16)).reshape(n, _HEADS * _HID) + b1)

    Cmu_s = W_mu @ a_src_mu[0]
    Cmu_d = W_mu @ a_dst_mu[0]
    Cls_s = W_ls @ a_src_ls[0]
    Cls_d = W_ls @ a_dst_ls[0]
    W2cat = jnp.concatenate(
        [W_mu, W_ls,
         Cmu_s[:, None], Cls_s[:, None], Cmu_d[:, None], Cls_d[:, None]],
        axis=1)
    out2 = _pl_matmul(h1, W2cat)
    sd2 = out2[:, 2 * _LAT:]

    tabs2, tabd2, g2 = score_prep(sd2[:, 0:2], sd2[:, 2:4], 2)
    eeh2 = _sc_scores(tabs2, tabd2, g2, srcs, dsts, 2)
    src4 = srcp.reshape(2 * _NT, _NSB2, _SB2, _CH)
    dst4 = dstp.reshape(2 * _NT, _NSB2, _SB2, _CH)
    eem4 = eeh2[0].reshape(2 * _NT, _NSB2, _EPS2)
    eel4 = eeh2[1].reshape(2 * _NT, _NSB2, _EPS2)
    o0, o1, dm, dl = _sc_agg2(out2[:, :2 * _LAT], eem4, eel4, src4, dst4,
                              zeros, zeros_np)
    out2agg = (o0 + o1)[:n]
    denm = dm.sum(axis=0)[:n, None]
    denl = dl.sum(axis=0)[:n, None]
    mu = out2agg[:, :_LAT] / (denm + 1e-16) + b_mu
    logstd = out2agg[:, _LAT:] / (denl + 1e-16) + b_ls
    return (mu, mu, logstd)
